# initial kernel scaffold (unmeasured)
import jax
import jax.numpy as jnp
from jax import lax
from jax.experimental import pallas as pl
from jax.experimental.pallas import tpu as pltpu

N_DEV = 4
B, Sq, Hq, Dh = 2, 512, 8, 64
SKV = 512
D_MODEL = 768
D_QK = Hq * Dh
SCALE = 0.125
NEG = -1e9


def kernel(x, Wq, K_ext, V_ext, Wo):
    def body(x_ref, wq_ref, k_ref, v_ref, wo_ref, out_ref,
             q_ref, kv_ref, acc_ref, m_ref, l_ref, ctx_ref,
             send_sems, recv_sems):
        my = lax.axis_index("i")
        left = lax.rem(my + N_DEV - 1, N_DEV)
        right = lax.rem(my + 1, N_DEV)

        for b in range(B):
            for hd in range(Hq):
                kv_ref[0, 0, b, hd] = k_ref[b, :, hd, :].astype(jnp.bfloat16)
                kv_ref[0, 1, b, hd] = v_ref[b, :, hd, :].astype(jnp.bfloat16)

        wq = wq_ref[...].astype(jnp.bfloat16)
        for b in range(B):
            qb = lax.dot_general(
                x_ref[b].astype(jnp.bfloat16), wq,
                (((1,), (0,)), ((), ())),
                preferred_element_type=jnp.float32,
            ).astype(jnp.bfloat16)
            for hd in range(Hq):
                q_ref[b, hd] = qb[:, hd * Dh:(hd + 1) * Dh]

        acc_ref[...] = jnp.zeros((B, Hq, Sq, Dh), jnp.float32)
        m_ref[...] = jnp.full((B, Hq, Sq, 1), -1e30, jnp.float32)
        l_ref[...] = jnp.zeros((B, Hq, Sq, 1), jnp.float32)

        barrier = pltpu.get_barrier_semaphore()
        for nbr in (left, right):
            pl.semaphore_signal(barrier, inc=1, device_id=(nbr,),
                                device_id_type=pl.DeviceIdType.MESH)
        pl.semaphore_wait(barrier, 2)

        qi = lax.broadcasted_iota(jnp.int32, (Sq, SKV), 0)
        kj = lax.broadcasted_iota(jnp.int32, (Sq, SKV), 1)

        def process_chunk(slot):
            origin = lax.rem(my + (N_DEV - slot), N_DEV)
            off = origin * SKV
            kg = kj + off
            mask = (jnp.abs(qi - kg) <= 128) | (kg < 32) | (qi < 32)

            def one_head(i, _):
                b = i // Hq
                hd = lax.rem(i, Hq)
                q = q_ref[b, hd]
                k = kv_ref[slot, 0, b, hd]
                s = lax.dot_general(
                    q, k, (((1,), (1,)), ((), ())),
                    preferred_element_type=jnp.float32,
                ) * SCALE
                s = jnp.where(mask, s, NEG)
                m_old = m_ref[b, hd]
                m_new = jnp.maximum(m_old, jnp.max(s, axis=1, keepdims=True))
                alpha = jnp.exp(m_old - m_new)
                p = jnp.exp(s - m_new)
                l_ref[b, hd] = alpha * l_ref[b, hd] + jnp.sum(
                    s=None, a=p, axis=1, keepdims=True)
                v = kv_ref[slot, 1, b, hd]
                pv = lax.dot_general(
                    p.astype(jnp.bfloat16), v, (((1,), (0,)), ((), ())),
                    preferred_element_type=jnp.float32,
                )
                acc_ref[b, hd] = acc_ref[b, hd] * alpha + pv
                m_ref[b, hd] = m_new
                return 0

            lax.fori_loop(0, B * Hq, one_head, 0)

        for h in range(N_DEV - 1):
            rdma = pltpu.make_async_remote_copy(
                src_ref=kv_ref.at[h],
                dst_ref=kv_ref.at[h + 1],
                send_sem=send_sems.at[h],
                recv_sem=recv_sems.at[h],
                device_id=(right,),
                device_id_type=pl.DeviceIdType.MESH,
            )
            rdma.start()
            process_chunk(h)
            rdma.wait()
        process_chunk(N_DEV - 1)

        wo = wo_ref[...].astype(jnp.bfloat16)
        for b in range(B):
            for hd in range(Hq):
                ctx_ref[b, :, hd * Dh:(hd + 1) * Dh] = (
                    acc_ref[b, hd] / l_ref[b, hd]).astype(jnp.bfloat16)
            out_ref[b] = lax.dot_general(
                ctx_ref[b], wo, (((1,), (0,)), ((), ())),
                preferred_element_type=jnp.float32,
            )

    return pl.pallas_call(
        body,
        out_shape=jax.ShapeDtypeStruct((B, Sq, D_MODEL), jnp.float32),
        in_specs=[pl.BlockSpec(memory_space=pltpu.VMEM)] * 5,
        out_specs=pl.BlockSpec(memory_space=pltpu.VMEM),
        scratch_shapes=[
            pltpu.VMEM((B, Hq, Sq, Dh), jnp.bfloat16),
            pltpu.VMEM((N_DEV, 2, B, Hq, SKV, Dh), jnp.bfloat16),
            pltpu.VMEM((B, Hq, Sq, Dh), jnp.float32),
            pltpu.VMEM((B, Hq, Sq, 1), jnp.float32),
            pltpu.VMEM((B, Hq, Sq, 1), jnp.float32),
            pltpu.VMEM((B, Sq, D_QK), jnp.bfloat16),
            pltpu.SemaphoreType.DMA((N_DEV - 1,)),
            pltpu.SemaphoreType.DMA((N_DEV - 1,)),
        ],
        compiler_params=pltpu.CompilerParams(collective_id=0),
    )(x, Wq, K_ext, V_ext, Wo)


# baseline (device time: 116475 ns/iter reference)
import jax
import jax.numpy as jnp
from jax import lax
from jax.experimental import pallas as pl
from jax.experimental.pallas import tpu as pltpu

N_DEV = 4
B, Sq, Hq, Dh = 2, 512, 8, 64
SKV = 512
D_MODEL = 768
D_QK = Hq * Dh
SCALE = 0.125
NEG = -1e9


def kernel(x, Wq, K_ext, V_ext, Wo):
    xb = x.astype(jnp.bfloat16)
    wqb = Wq.astype(jnp.bfloat16)
    wob = Wo.astype(jnp.bfloat16)
    kv = jnp.stack([
        K_ext.reshape(B, SKV, D_QK),
        V_ext.reshape(B, SKV, D_QK),
    ]).astype(jnp.bfloat16)

    def body(x_ref, wq_ref, kv_ref, wo_ref, out_ref,
             q_ref, comm_ref, acc_ref, m_ref, l_ref, ctx_ref,
             send_sems, recv_sems):
        my = lax.axis_index("i")
        left = lax.rem(my + N_DEV - 1, N_DEV)
        right = lax.rem(my + 1, N_DEV)

        for b in range(B):
            q_ref[b] = lax.dot_general(
                x_ref[b], wq_ref[...],
                (((1,), (0,)), ((), ())),
                preferred_element_type=jnp.float32,
            ).astype(jnp.bfloat16)

        acc_ref[...] = jnp.zeros((B, Sq, D_QK), jnp.float32)
        m_ref[...] = jnp.full((B, Sq, Hq), -1e30, jnp.float32)
        l_ref[...] = jnp.zeros((B, Sq, Hq), jnp.float32)

        barrier = pltpu.get_barrier_semaphore()
        for nbr in (left, right):
            pl.semaphore_signal(barrier, inc=1, device_id=(nbr,),
                                device_id_type=pl.DeviceIdType.MESH)
        pl.semaphore_wait(barrier, 2)

        def process_chunk(src_ref, origin):
            off = origin * SKV
            qi = lax.broadcasted_iota(jnp.int32, (Sq, SKV), 0)
            kg = lax.broadcasted_iota(jnp.int32, (Sq, SKV), 1) + off
            mask = (jnp.abs(qi - kg) <= 128) | (kg < 32) | (qi < 32)
            for b in range(B):
                for hd in range(Hq):
                    sl = slice(hd * Dh, (hd + 1) * Dh)
                    q = q_ref[b, :, sl]
                    s = lax.dot_general(
                        q, src_ref[0, b, :, sl], (((1,), (1,)), ((), ())),
                        preferred_element_type=jnp.float32,
                    ) * SCALE
                    s = jnp.where(mask, s, NEG)
                    m_old = m_ref[b, :, hd:hd + 1]
                    m_new = jnp.maximum(
                        m_old, jnp.max(s, axis=1, keepdims=True))
                    alpha = jnp.exp(m_old - m_new)
                    p = jnp.exp(s - m_new)
                    l_ref[b, :, hd:hd + 1] = (
                        alpha * l_ref[b, :, hd:hd + 1]
                        + jnp.sum(p, axis=1, keepdims=True))
                    pv = lax.dot_general(
                        p.astype(jnp.bfloat16), src_ref[1, b, :, sl],
                        (((1,), (0,)), ((), ())),
                        preferred_element_type=jnp.float32,
                    )
                    acc_ref[b, :, sl] = acc_ref[b, :, sl] * alpha + pv
                    m_ref[b, :, hd:hd + 1] = m_new

        for h in range(N_DEV - 1):
            src = kv_ref if h == 0 else comm_ref.at[h - 1]
            rdma = pltpu.make_async_remote_copy(
                src_ref=src,
                dst_ref=comm_ref.at[h],
                send_sem=send_sems.at[h],
                recv_sem=recv_sems.at[h],
                device_id=(right,),
                device_id_type=pl.DeviceIdType.MESH,
            )
            rdma.start()
            process_chunk(src, lax.rem(my + N_DEV - h, N_DEV))
            rdma.wait()
        process_chunk(comm_ref.at[N_DEV - 2], lax.rem(my + 1, N_DEV))

        for b in range(B):
            for hd in range(Hq):
                sl = slice(hd * Dh, (hd + 1) * Dh)
                ctx_ref[b, :, sl] = (
                    acc_ref[b, :, sl] / l_ref[b, :, hd:hd + 1]
                ).astype(jnp.bfloat16)
            out_ref[b] = lax.dot_general(
                ctx_ref[b], wo_ref[...], (((1,), (0,)), ((), ())),
                preferred_element_type=jnp.float32,
            )

    return pl.pallas_call(
        body,
        out_shape=jax.ShapeDtypeStruct((B, Sq, D_MODEL), jnp.float32),
        in_specs=[pl.BlockSpec(memory_space=pltpu.VMEM)] * 4,
        out_specs=pl.BlockSpec(memory_space=pltpu.VMEM),
        scratch_shapes=[
            pltpu.VMEM((B, Sq, D_QK), jnp.bfloat16),
            pltpu.VMEM((N_DEV - 1, 2, B, SKV, D_QK), jnp.bfloat16),
            pltpu.VMEM((B, Sq, D_QK), jnp.float32),
            pltpu.VMEM((B, Sq, Hq), jnp.float32),
            pltpu.VMEM((B, Sq, Hq), jnp.float32),
            pltpu.VMEM((B, Sq, D_QK), jnp.bfloat16),
            pltpu.SemaphoreType.DMA((N_DEV - 1,)),
            pltpu.SemaphoreType.DMA((N_DEV - 1,)),
        ],
        compiler_params=pltpu.CompilerParams(
            collective_id=0, vmem_limit_bytes=100 * 1024 * 1024),
    )(xb, wqb, kv, wob)


# device time: 57853 ns/iter; 2.0133x vs baseline; 2.0133x over previous
import jax
import jax.numpy as jnp
from jax import lax
from jax.experimental import pallas as pl
from jax.experimental.pallas import tpu as pltpu

N_DEV = 4
B, Sq, Hq, Dh = 2, 512, 8, 64
SKV = 512
D_MODEL = 768
D_QK = Hq * Dh
SCALE = 0.125


def kernel(x, Wq, K_ext, V_ext, Wo):
    xb = x.astype(jnp.bfloat16)
    wqb = Wq.astype(jnp.bfloat16)
    wob = Wo.astype(jnp.bfloat16)
    kv = jnp.stack([
        K_ext.reshape(B, SKV, D_QK),
        V_ext.reshape(B, SKV, D_QK),
    ]).astype(jnp.bfloat16)

    def body(x_ref, wq_ref, kv_ref, wo_ref, out_ref,
             q_ref, st_ref, l_ref, stL_ref, lL_ref, stR_ref, lR_ref,
             stD_ref, lD_ref, ctx_ref, send_sems, recv_sems):
        my = lax.axis_index("i")
        left = lax.rem(my + N_DEV - 1, N_DEV)
        right = lax.rem(my + 1, N_DEV)

        for b in range(B):
            q_ref[b] = lax.dot_general(
                x_ref[b], wq_ref[...],
                (((1,), (0,)), ((), ())),
                preferred_element_type=jnp.float32,
            ).astype(jnp.bfloat16)

        off = my * SKV
        qi = lax.broadcasted_iota(jnp.int32, (Sq, SKV), 0)
        kg = lax.broadcasted_iota(jnp.int32, (Sq, SKV), 1) + off
        mask = (jnp.abs(qi - kg) <= 128) | (kg < 32) | (qi < 32)
        for b in range(B):
            for hd in range(Hq):
                sl = slice(hd * Dh, (hd + 1) * Dh)
                s = lax.dot_general(
                    q_ref[b, :, sl], kv_ref[0, b, :, sl],
                    (((1,), (1,)), ((), ())),
                    preferred_element_type=jnp.float32,
                ) * SCALE
                p = jnp.where(mask, jnp.exp(s), 0.0)
                l_ref[b, :, hd:hd + 1] = jnp.sum(p, axis=1, keepdims=True)
                st_ref[b, :, sl] = lax.dot_general(
                    p.astype(jnp.bfloat16), kv_ref[1, b, :, sl],
                    (((1,), (0,)), ((), ())),
                    preferred_element_type=jnp.float32,
                ).astype(jnp.bfloat16)

        barrier = pltpu.get_barrier_semaphore()
        for nbr in (left, right):
            pl.semaphore_signal(barrier, inc=1, device_id=(nbr,),
                                device_id_type=pl.DeviceIdType.MESH)
        pl.semaphore_wait(barrier, 2)

        def rdma(i, src, dst, dev):
            return pltpu.make_async_remote_copy(
                src_ref=src, dst_ref=dst,
                send_sem=send_sems.at[i], recv_sem=recv_sems.at[i],
                device_id=(dev,), device_id_type=pl.DeviceIdType.MESH,
            )

        hop1 = [
            rdma(0, st_ref, stL_ref, right),
            rdma(1, l_ref, lL_ref, right),
            rdma(2, st_ref, stR_ref, left),
            rdma(3, l_ref, lR_ref, left),
        ]
        for r in hop1:
            r.start()
        for r in hop1:
            r.wait()

        hop2 = [
            rdma(4, stL_ref.at[0:1], stD_ref.at[0:1], right),
            rdma(5, lL_ref.at[0:1], lD_ref.at[0:1], right),
            rdma(6, stR_ref.at[1:2], stD_ref.at[1:2], left),
            rdma(7, lR_ref.at[1:2], lD_ref.at[1:2], left),
        ]
        for r in hop2:
            r.start()
        for r in hop2:
            r.wait()

        for b in range(B):
            acc = (st_ref[b].astype(jnp.float32)
                   + stL_ref[b].astype(jnp.float32)
                   + stR_ref[b].astype(jnp.float32)
                   + stD_ref[b].astype(jnp.float32))
            l_tot = l_ref[b] + lL_ref[b] + lR_ref[b] + lD_ref[b]
            for hd in range(Hq):
                sl = slice(hd * Dh, (hd + 1) * Dh)
                ctx_ref[b, :, sl] = (
                    acc[:, sl] / l_tot[:, hd:hd + 1]
                ).astype(jnp.bfloat16)
            out_ref[b] = lax.dot_general(
                ctx_ref[b], wo_ref[...], (((1,), (0,)), ((), ())),
                preferred_element_type=jnp.float32,
            )

    st_shape = pltpu.VMEM((B, Sq, D_QK), jnp.bfloat16)
    l_shape = pltpu.VMEM((B, Sq, Hq), jnp.float32)
    return pl.pallas_call(
        body,
        out_shape=jax.ShapeDtypeStruct((B, Sq, D_MODEL), jnp.float32),
        in_specs=[pl.BlockSpec(memory_space=pltpu.VMEM)] * 4,
        out_specs=pl.BlockSpec(memory_space=pltpu.VMEM),
        scratch_shapes=[
            pltpu.VMEM((B, Sq, D_QK), jnp.bfloat16),
            st_shape, l_shape,
            st_shape, l_shape,
            st_shape, l_shape,
            st_shape, l_shape,
            pltpu.VMEM((B, Sq, D_QK), jnp.bfloat16),
            pltpu.SemaphoreType.DMA((8,)),
            pltpu.SemaphoreType.DMA((8,)),
        ],
        compiler_params=pltpu.CompilerParams(
            collective_id=0, vmem_limit_bytes=100 * 1024 * 1024),
    )(xb, wqb, kv, wob)
